# baseline (speedup 1.0000x reference)
import jax
import jax.numpy as jnp
from jax.experimental import pallas as pl
from jax.experimental.pallas import tpu as pltpu

_B1 = 2048
_B2 = 8192


def _stats_kernel(s_ref, y_ref, w1_ref, b1_ref, w2_ref, b2_ref, w3_ref, b3_ref,
                  m_out_ref, mx_ref, d_ref, n_ref):
    i = pl.program_id(0)
    nb = pl.num_programs(0)

    @pl.when(i == 0)
    def _init():
        mx_ref[...] = jnp.full_like(mx_ref, -1e30)
        d_ref[...] = jnp.zeros_like(d_ref)
        n_ref[...] = jnp.zeros_like(n_ref)

    s = s_ref[...]
    y = y_ref[...]
    yb = y.astype(jnp.bfloat16)
    h = jnp.maximum(jnp.dot(yb, w1_ref[...],
                            preferred_element_type=jnp.float32) + b1_ref[...], 0.0)
    h = jnp.maximum(jnp.dot(h.astype(jnp.bfloat16), w2_ref[...],
                            preferred_element_type=jnp.float32) + b2_ref[...], 0.0)
    logits = jnp.dot(h.astype(jnp.bfloat16), w3_ref[...],
                     preferred_element_type=jnp.float32) + b3_ref[...]

    p = s.shape[1]
    c = jnp.argmax(s, axis=1)
    onehot = (c[:, None] == jax.lax.broadcasted_iota(jnp.int32, (1, p), 1)
              ).astype(jnp.bfloat16)

    m_old = mx_ref[...]
    m_new = jnp.maximum(m_old, jnp.max(logits, axis=0, keepdims=True))
    scale = jnp.exp(m_old - m_new)
    e = jnp.exp(logits - m_new)
    eb = e.astype(jnp.bfloat16)
    eyb = (e * y).astype(jnp.bfloat16)
    d_blk = jax.lax.dot_general(onehot, eb, (((0,), (0,)), ((), ())),
                                preferred_element_type=jnp.float32)
    n_blk = jax.lax.dot_general(onehot, eyb, (((0,), (0,)), ((), ())),
                                preferred_element_type=jnp.float32)
    mx_ref[...] = m_new
    d_ref[...] = d_ref[...] * scale + d_blk
    n_ref[...] = n_ref[...] * scale + n_blk

    @pl.when(i == nb - 1)
    def _finalize():
        dd = d_ref[...]
        m_out_ref[...] = jnp.where(
            dd > 0, n_ref[...] / jnp.maximum(dd, 1e-30), 0.0)


def _combine_kernel(s_ref, m_ref, out_ref):
    out_ref[...] = jnp.dot(s_ref[...], m_ref[...],
                           preferred_element_type=jnp.float32)


def kernel(abundance_matrix, Y, W1, b1, W2, b2, W3, b3):
    n, p = abundance_matrix.shape
    f = Y.shape[1]
    h = W1.shape[1]
    w1b = W1.astype(jnp.bfloat16)
    w2b = W2.astype(jnp.bfloat16)
    w3b = W3.astype(jnp.bfloat16)
    b1r = b1.reshape(1, h)
    b2r = b2.reshape(1, h)
    b3r = b3.reshape(1, f)

    nb1 = n // _B1
    M = pl.pallas_call(
        _stats_kernel,
        grid=(nb1,),
        in_specs=[
            pl.BlockSpec((_B1, p), lambda i: (i, 0)),
            pl.BlockSpec((_B1, f), lambda i: (i, 0)),
            pl.BlockSpec((W1.shape[0], h), lambda i: (0, 0)),
            pl.BlockSpec((1, h), lambda i: (0, 0)),
            pl.BlockSpec((h, h), lambda i: (0, 0)),
            pl.BlockSpec((1, h), lambda i: (0, 0)),
            pl.BlockSpec((h, f), lambda i: (0, 0)),
            pl.BlockSpec((1, f), lambda i: (0, 0)),
        ],
        out_specs=pl.BlockSpec((p, f), lambda i: (0, 0)),
        out_shape=jax.ShapeDtypeStruct((p, f), jnp.float32),
        scratch_shapes=[
            pltpu.VMEM((1, f), jnp.float32),
            pltpu.VMEM((p, f), jnp.float32),
            pltpu.VMEM((p, f), jnp.float32),
        ],
    )(abundance_matrix, Y, w1b, b1r, w2b, b2r, w3b, b3r)

    nb2 = n // _B2
    Y_hat = pl.pallas_call(
        _combine_kernel,
        grid=(nb2,),
        in_specs=[
            pl.BlockSpec((_B2, p), lambda i: (i, 0)),
            pl.BlockSpec((p, f), lambda i: (0, 0)),
        ],
        out_specs=pl.BlockSpec((_B2, f), lambda i: (i, 0)),
        out_shape=jax.ShapeDtypeStruct((n, f), jnp.float32),
    )(abundance_matrix, M)
    return Y_hat


# speedup vs baseline: 1.0764x; 1.0764x over previous
import jax
import jax.numpy as jnp
from jax.experimental import pallas as pl
from jax.experimental.pallas import tpu as pltpu

_B = 4096


def _fused_kernel(s_ref, y_ref, w1_ref, b1_ref, w2_ref, b2_ref, w3_ref, b3_ref,
                  out_ref, st_ref, mx_ref, d_ref, n_ref, mfin_ref):
    g = pl.program_id(0)
    nb = pl.num_programs(0) // 2

    @pl.when(g == 0)
    def _init():
        mx_ref[...] = jnp.full_like(mx_ref, -1e30)
        d_ref[...] = jnp.zeros_like(d_ref)
        n_ref[...] = jnp.zeros_like(n_ref)

    @pl.when(g < nb)
    def _stats_phase():
        s = s_ref[...]
        y = y_ref[...]
        b = s.shape[0]
        p = s.shape[1]

        st_ref[:, pl.ds(g * b, b)] = jnp.swapaxes(s, 0, 1)

        yb = y.astype(jnp.bfloat16)
        h = jnp.maximum(jnp.dot(yb, w1_ref[...],
                                preferred_element_type=jnp.float32)
                        + b1_ref[...], 0.0)
        h = jnp.maximum(jnp.dot(h.astype(jnp.bfloat16), w2_ref[...],
                                preferred_element_type=jnp.float32)
                        + b2_ref[...], 0.0)
        logits = jnp.dot(h.astype(jnp.bfloat16), w3_ref[...],
                         preferred_element_type=jnp.float32) + b3_ref[...]

        c = jnp.argmax(s, axis=1)
        onehot = (c[:, None] == jax.lax.broadcasted_iota(jnp.int32, (1, p), 1)
                  ).astype(jnp.bfloat16)

        m_old = mx_ref[...]
        m_new = jnp.maximum(m_old, jnp.max(logits, axis=0, keepdims=True))
        scale = jnp.exp(m_old - m_new)
        e = jnp.exp(logits - m_new)
        d_blk = jax.lax.dot_general(onehot, e.astype(jnp.bfloat16),
                                    (((0,), (0,)), ((), ())),
                                    preferred_element_type=jnp.float32)
        n_blk = jax.lax.dot_general(onehot, (e * y).astype(jnp.bfloat16),
                                    (((0,), (0,)), ((), ())),
                                    preferred_element_type=jnp.float32)
        mx_ref[...] = m_new
        d_ref[...] = d_ref[...] * scale + d_blk
        n_ref[...] = n_ref[...] * scale + n_blk

        @pl.when(g == nb - 1)
        def _finalize():
            dd = d_ref[...]
            mfin_ref[...] = jnp.where(
                dd > 0, n_ref[...] / jnp.maximum(dd, 1e-30), 0.0)

    @pl.when(g >= nb)
    def _combine_phase():
        b = out_ref.shape[0]
        st = st_ref[:, pl.ds((g - nb) * b, b)]
        out_ref[...] = jax.lax.dot_general(
            st, mfin_ref[...], (((0,), (0,)), ((), ())),
            preferred_element_type=jnp.float32)


def kernel(abundance_matrix, Y, W1, b1, W2, b2, W3, b3):
    n, p = abundance_matrix.shape
    f = Y.shape[1]
    h = W1.shape[1]
    w1b = W1.astype(jnp.bfloat16)
    w2b = W2.astype(jnp.bfloat16)
    w3b = W3.astype(jnp.bfloat16)
    b1r = b1.reshape(1, h)
    b2r = b2.reshape(1, h)
    b3r = b3.reshape(1, f)

    nb = n // _B
    clamp = lambda g: (jnp.minimum(g, nb - 1), 0)
    Y_hat = pl.pallas_call(
        _fused_kernel,
        grid=(2 * nb,),
        in_specs=[
            pl.BlockSpec((_B, p), clamp),
            pl.BlockSpec((_B, f), clamp),
            pl.BlockSpec((W1.shape[0], h), lambda g: (0, 0)),
            pl.BlockSpec((1, h), lambda g: (0, 0)),
            pl.BlockSpec((h, h), lambda g: (0, 0)),
            pl.BlockSpec((1, h), lambda g: (0, 0)),
            pl.BlockSpec((h, f), lambda g: (0, 0)),
            pl.BlockSpec((1, f), lambda g: (0, 0)),
        ],
        out_specs=pl.BlockSpec((_B, f), lambda g: (jnp.maximum(g - nb, 0), 0)),
        out_shape=jax.ShapeDtypeStruct((n, f), jnp.float32),
        scratch_shapes=[
            pltpu.VMEM((p, n), jnp.float32),
            pltpu.VMEM((1, f), jnp.float32),
            pltpu.VMEM((p, f), jnp.float32),
            pltpu.VMEM((p, f), jnp.float32),
            pltpu.VMEM((p, f), jnp.float32),
        ],
    )(abundance_matrix, Y, w1b, b1r, w2b, b2r, w3b, b3r)
    return Y_hat
